# Initial kernel scaffold; baseline (speedup 1.0000x reference)
import jax
import jax.numpy as jnp
from jax.experimental import pallas as pl


def kernel(x, w):
    raise NotImplementedError("write your pallas kernel here")


# speedup vs baseline: 4.7485x; 4.7485x over previous
import functools

import jax
import jax.numpy as jnp
from jax import lax
from jax.experimental import pallas as pl
from jax.experimental.pallas import tpu as pltpu
from jax.experimental.pallas import tpu_sc as plsc


@functools.lru_cache(maxsize=None)
def _make_gather(V, D, B):
    info = plsc.get_sparse_core_info()
    NC, NS = info.num_cores, info.num_subcores
    NW = NC * NS
    K = 4
    CHUNK = K * 128
    rows_per_w = B // 128 // NW
    n_chunks = rows_per_w // K
    mesh = plsc.VectorSubcoreMesh(core_axis_name="c", subcore_axis_name="s")

    @functools.partial(
        pl.kernel,
        mesh=mesh,
        compiler_params=pltpu.CompilerParams(use_tc_tiling_on_sc=False),
        out_type=jax.ShapeDtypeStruct((B, D), jnp.float32),
        scratch_types=[
            pltpu.VMEM((K, 128), jnp.int32),
            pltpu.VMEM((CHUNK, D), jnp.float32),
            pltpu.SemaphoreType.DMA,
        ],
    )
    def gather(table_hbm, idx_hbm, out_hbm, idx_v, rows_v, sem):
        wid = lax.axis_index("s") * NC + lax.axis_index("c")
        row0 = wid * rows_per_w

        def body(i, carry):
            r = row0 + i * K
            pltpu.sync_copy(idx_hbm.at[pl.ds(r, K)], idx_v)
            cps = [
                pltpu.async_copy(
                    table_hbm.at[idx_v.at[j]],
                    rows_v.at[pl.ds(j * 128, 128)],
                    sem,
                )
                for j in range(K)
            ]
            for cp in cps:
                cp.wait()
            pltpu.sync_copy(rows_v, out_hbm.at[pl.ds(r * 128, CHUNK)])
            return carry

        lax.fori_loop(0, n_chunks, body, 0)

    return gather


def kernel(x, w):
    B0, H = x.shape
    V, D = w.shape
    B = B0 * H
    idx2d = x.reshape(B // 128, 128)
    out = _make_gather(V, D, B)(w, idx2d)
    return jax.lax.stop_gradient(out.reshape(B0, H, D))


# speedup vs baseline: 5.1583x; 1.0863x over previous
import functools

import jax
import jax.numpy as jnp
from jax import lax
from jax.experimental import pallas as pl
from jax.experimental.pallas import tpu as pltpu
from jax.experimental.pallas import tpu_sc as plsc


@functools.lru_cache(maxsize=None)
def _make_gather(V, D, B):
    info = plsc.get_sparse_core_info()
    NC, NS = info.num_cores, info.num_subcores
    NW = NC * NS
    K = 4
    CHUNK = K * 128
    rows_per_w = B // 128 // NW
    n_chunks = rows_per_w // K
    assert n_chunks % 2 == 0 and n_chunks >= 4
    mesh = plsc.VectorSubcoreMesh(core_axis_name="c", subcore_axis_name="s")

    @functools.partial(
        pl.kernel,
        mesh=mesh,
        compiler_params=pltpu.CompilerParams(use_tc_tiling_on_sc=False),
        out_type=jax.ShapeDtypeStruct((B, D), jnp.float32),
        scratch_types=[
            pltpu.VMEM((2, K, 128), jnp.int32),
            pltpu.VMEM((2, CHUNK, D), jnp.float32),
            pltpu.SemaphoreType.DMA,
            pltpu.SemaphoreType.DMA,
        ],
    )
    def gather(table_hbm, idx_hbm, out_hbm, idx_v, rows_v, gsem, ssem):
        wid = lax.axis_index("s") * NC + lax.axis_index("c")
        row0 = wid * rows_per_w

        def fire_gathers(c, b):
            pltpu.sync_copy(idx_hbm.at[pl.ds(row0 + c * K, K)], idx_v.at[b])
            for j in range(K):
                pltpu.async_copy(
                    table_hbm.at[idx_v.at[b].at[j]],
                    rows_v.at[b].at[pl.ds(j * 128, 128)],
                    gsem,
                )

        def wait_gathers(b):
            pltpu.make_async_copy(
                table_hbm.at[pl.ds(0, CHUNK)], rows_v.at[b], gsem
            ).wait()

        def fire_store(c, b):
            pltpu.async_copy(
                rows_v.at[b],
                out_hbm.at[pl.ds((row0 + c * K) * 128, CHUNK)],
                ssem,
            )

        def wait_store(b):
            pltpu.make_async_copy(
                rows_v.at[b], out_hbm.at[pl.ds(0, CHUNK)], ssem
            ).wait()

        fire_gathers(0, 0)
        fire_gathers(1, 1)
        wait_gathers(0)
        fire_store(0, 0)
        wait_gathers(1)
        fire_store(1, 1)

        def body(g, carry):
            c0 = 2 + 2 * g
            wait_store(0)
            fire_gathers(c0, 0)
            wait_store(1)
            fire_gathers(c0 + 1, 1)
            wait_gathers(0)
            fire_store(c0, 0)
            wait_gathers(1)
            fire_store(c0 + 1, 1)
            return carry

        lax.fori_loop(0, (n_chunks - 2) // 2, body, 0)
        wait_store(0)
        wait_store(1)

    return gather


def kernel(x, w):
    B0, H = x.shape
    V, D = w.shape
    B = B0 * H
    idx2d = x.reshape(B // 128, 128)
    out = _make_gather(V, D, B)(w, idx2d)
    return jax.lax.stop_gradient(out.reshape(B0, H, D))


# speedup vs baseline: 5.1663x; 1.0016x over previous
import functools

import jax
import jax.numpy as jnp
from jax import lax
from jax.experimental import pallas as pl
from jax.experimental.pallas import tpu as pltpu
from jax.experimental.pallas import tpu_sc as plsc


@functools.lru_cache(maxsize=None)
def _make_gather(V, D, B0, H):
    info = plsc.get_sparse_core_info()
    NC, NS = info.num_cores, info.num_subcores
    NW = NC * NS
    K = 4
    CHUNK = K * H
    slabs_per_w = B0 // NW
    n_chunks = slabs_per_w // K
    assert n_chunks % 2 == 0 and n_chunks >= 4
    pieces = []
    off = 0
    while off < H:
        ln = min(128, H - off)
        pieces.append((off, ln))
        off += ln
    mesh = plsc.VectorSubcoreMesh(core_axis_name="c", subcore_axis_name="s")

    @functools.partial(
        pl.kernel,
        mesh=mesh,
        compiler_params=pltpu.CompilerParams(use_tc_tiling_on_sc=False),
        out_type=jax.ShapeDtypeStruct((B0, H, D), jnp.float32),
        scratch_types=[
            pltpu.VMEM((2, K, H), jnp.int32),
            pltpu.VMEM((2, K, H, D), jnp.float32),
            pltpu.SemaphoreType.DMA,
            pltpu.SemaphoreType.DMA,
        ],
    )
    def gather(table_hbm, idx_hbm, out_hbm, idx_v, rows_v, gsem, ssem):
        wid = lax.axis_index("s") * NC + lax.axis_index("c")
        slab0 = wid * slabs_per_w

        def fire_gathers(c, b):
            pltpu.sync_copy(idx_hbm.at[pl.ds(slab0 + c * K, K)], idx_v.at[b])
            for s in range(K):
                for off, ln in pieces:
                    pltpu.async_copy(
                        table_hbm.at[idx_v.at[b].at[s].at[pl.ds(off, ln)]],
                        rows_v.at[b].at[s].at[pl.ds(off, ln)],
                        gsem,
                    )

        def wait_gathers_all(b):
            for s in range(K):
                for off, ln in pieces:
                    pltpu.make_async_copy(
                        table_hbm.at[pl.ds(0, ln)],
                        rows_v.at[b].at[s].at[pl.ds(off, ln)],
                        gsem,
                    ).wait()

        def fire_store(c, b):
            pltpu.async_copy(
                rows_v.at[b],
                out_hbm.at[pl.ds(slab0 + c * K, K)],
                ssem,
            )

        def wait_store(b):
            pltpu.make_async_copy(
                rows_v.at[b], out_hbm.at[pl.ds(0, K)], ssem
            ).wait()

        fire_gathers(0, 0)
        fire_gathers(1, 1)
        wait_gathers_all(0)
        fire_store(0, 0)
        wait_gathers_all(1)
        fire_store(1, 1)

        def body(g, carry):
            c0 = 2 + 2 * g
            wait_store(0)
            fire_gathers(c0, 0)
            wait_store(1)
            fire_gathers(c0 + 1, 1)
            wait_gathers_all(0)
            fire_store(c0, 0)
            wait_gathers_all(1)
            fire_store(c0 + 1, 1)
            return carry

        lax.fori_loop(0, (n_chunks - 2) // 2, body, 0)
        wait_store(0)
        wait_store(1)

    return gather


def kernel(x, w):
    B0, H = x.shape
    V, D = w.shape
    out = _make_gather(V, D, B0, H)(w, x)
    return jax.lax.stop_gradient(out)
